# baseline (speedup 1.0000x reference)
import functools

import jax
import jax.numpy as jnp
from jax import lax
from jax.experimental import pallas as pl
from jax.experimental.pallas import tpu as pltpu
from jax.experimental.pallas import tpu_sc as plsc

N = 10000
NPAD = 10240
NC, NS = 2, 16
NW = NC * NS
F = 16
RPS = NPAD // NS


def _sc_pass(table, src_r, dst_r, zeros):
    rows_pw = src_r.shape[1]
    mesh = plsc.VectorSubcoreMesh(core_axis_name="c", subcore_axis_name="s")

    @functools.partial(
        pl.kernel,
        out_type=jax.ShapeDtypeStruct((NC, NPAD, F), jnp.float32),
        mesh=mesh,
        compiler_params=pltpu.CompilerParams(use_tc_tiling_on_sc=False),
        scratch_types=[
            pltpu.VMEM((rows_pw, 128), jnp.int32),
            pltpu.VMEM((rows_pw, 128), jnp.int32),
            pltpu.VMEM((128, F), jnp.float32),
            pltpu.VMEM_SHARED((NPAD, F), jnp.float32),
        ],
    )
    def k(table_hbm, src_hbm, dst_hbm, z_hbm, out_hbm, src_v, dst_v, msg_v, acc):
        c = lax.axis_index("c")
        s = lax.axis_index("s")
        w = s * NC + c
        pltpu.sync_copy(src_hbm.at[w], src_v)
        pltpu.sync_copy(dst_hbm.at[w], dst_v)
        pltpu.sync_copy(z_hbm.at[pl.ds(s * RPS, RPS)], acc.at[pl.ds(s * RPS, RPS)])
        plsc.subcore_barrier()

        @pl.loop(0, rows_pw)
        def _(j):
            pltpu.sync_copy(table_hbm.at[src_v.at[j]], msg_v)
            pltpu.sync_copy(msg_v, acc.at[dst_v.at[j]], add=True)

        plsc.subcore_barrier()
        pltpu.sync_copy(acc.at[pl.ds(s * RPS, RPS)],
                        out_hbm.at[c].at[pl.ds(s * RPS, RPS)])

    return k(table, src_r, dst_r, zeros)


def _sc_deg_pass(ones, dst_r, zeros):
    rows_pw = dst_r.shape[1]
    mesh = plsc.VectorSubcoreMesh(core_axis_name="c", subcore_axis_name="s")

    @functools.partial(
        pl.kernel,
        out_type=jax.ShapeDtypeStruct((NC, NPAD, F), jnp.float32),
        mesh=mesh,
        compiler_params=pltpu.CompilerParams(use_tc_tiling_on_sc=False),
        scratch_types=[
            pltpu.VMEM((rows_pw, 128), jnp.int32),
            pltpu.VMEM((128, F), jnp.float32),
            pltpu.VMEM_SHARED((NPAD, F), jnp.float32),
        ],
    )
    def k(ones_hbm, dst_hbm, z_hbm, out_hbm, dst_v, ones_v, acc):
        c = lax.axis_index("c")
        s = lax.axis_index("s")
        w = s * NC + c
        pltpu.sync_copy(dst_hbm.at[w], dst_v)
        pltpu.sync_copy(ones_hbm.at[pl.ds(0, 128)], ones_v)
        pltpu.sync_copy(z_hbm.at[pl.ds(s * RPS, RPS)], acc.at[pl.ds(s * RPS, RPS)])
        plsc.subcore_barrier()

        @pl.loop(0, rows_pw)
        def _(j):
            pltpu.sync_copy(ones_v, acc.at[dst_v.at[j]], add=True)

        plsc.subcore_barrier()
        pltpu.sync_copy(acc.at[pl.ds(s * RPS, RPS)],
                        out_hbm.at[c].at[pl.ds(s * RPS, RPS)])

    return k(ones, dst_r, zeros)


_BLK = 1024
_GRID = NPAD // _BLK


def _row_spec(width):
    return pl.BlockSpec((_BLK, width), lambda i: (i, 0))


def _pair_spec(width):
    return pl.BlockSpec((NC, _BLK, width), lambda i: (0, i, 0))


def _full_spec(a, b):
    return pl.BlockSpec((a, b), lambda i: (0, 0))


def _elu(x):
    return jnp.where(x > 0, x, jnp.exp(jnp.minimum(x, 0.0)) - 1.0)


def _tc_xw1(xpad, W1):
    def body(x_ref, w_ref, o_ref):
        o_ref[...] = jnp.dot(x_ref[...], w_ref[...],
                             preferred_element_type=jnp.float32)

    return pl.pallas_call(
        body,
        grid=(_GRID,),
        in_specs=[_row_spec(128), _full_spec(128, F)],
        out_specs=_row_spec(F),
        out_shape=jax.ShapeDtypeStruct((NPAD, F), jnp.float32),
    )(xpad, W1)


def _tc_deg_finish(degp, xw1):
    def body(d_ref, xw_ref, dinv_ref, z_ref):
        deg = 1.0 + d_ref[0, :, 0:1] + d_ref[1, :, 0:1]
        dinv = lax.rsqrt(deg)
        dinv_ref[...] = jnp.broadcast_to(dinv, (_BLK, F))
        z_ref[...] = xw_ref[...] * dinv

    return pl.pallas_call(
        body,
        grid=(_GRID,),
        in_specs=[_pair_spec(F), _row_spec(F)],
        out_specs=[_row_spec(F), _row_spec(F)],
        out_shape=[jax.ShapeDtypeStruct((NPAD, F), jnp.float32),
                   jax.ShapeDtypeStruct((NPAD, F), jnp.float32)],
    )(degp, xw1)


def _tc_layer_mid(q, z, dinv16, b, Wn, scale_out):
    def body(q_ref, z_ref, dinv_ref, b_ref, w_ref, o_ref):
        u = q_ref[0] + q_ref[1] + z_ref[...]
        h = _elu(dinv_ref[...] * u + b_ref[...])
        o = jnp.dot(h, w_ref[...], preferred_element_type=jnp.float32)
        if scale_out:
            o = o * dinv_ref[...]
        o_ref[...] = o

    return pl.pallas_call(
        body,
        grid=(_GRID,),
        in_specs=[_pair_spec(F), _row_spec(F), _row_spec(F),
                  _full_spec(1, F), _full_spec(F, Wn.shape[1])],
        out_specs=_row_spec(Wn.shape[1]),
        out_shape=jax.ShapeDtypeStruct((NPAD, Wn.shape[1]), jnp.float32),
    )(q, z, dinv16, b, Wn)


def _tc_layer2_scale(q, z, dinv16, b):
    def body(q_ref, z_ref, dinv_ref, b_ref, o_ref):
        u = q_ref[0] + q_ref[1] + z_ref[...]
        h = _elu(dinv_ref[...] * u + b_ref[...])
        o_ref[...] = h * dinv_ref[...]

    return pl.pallas_call(
        body,
        grid=(_GRID,),
        in_specs=[_pair_spec(F), _row_spec(F), _row_spec(F), _full_spec(1, F)],
        out_specs=_row_spec(F),
        out_shape=jax.ShapeDtypeStruct((NPAD, F), jnp.float32),
    )(q, z, dinv16, b)


def _tc_final(q, g3, dinv16, b3, W3):
    K = W3.shape[1]

    def body(q_ref, g_ref, dinv_ref, b_ref, w_ref, o_ref):
        v = q_ref[0] + q_ref[1] + g_ref[...]
        m = jnp.dot(v, w_ref[...], preferred_element_type=jnp.float32)
        h = _elu(dinv_ref[:, 0:1] * m + b_ref[...])
        mx = jnp.max(h, axis=1, keepdims=True)
        e = jnp.exp(h - mx)
        lse = jnp.log(jnp.sum(e, axis=1, keepdims=True))
        o_ref[...] = h - mx - lse

    return pl.pallas_call(
        body,
        grid=(_GRID,),
        in_specs=[_pair_spec(F), _row_spec(F), _row_spec(F),
                  _full_spec(1, K), _full_spec(F, K)],
        out_specs=_row_spec(K),
        out_shape=jax.ShapeDtypeStruct((NPAD, K), jnp.float32),
    )(q, g3, dinv16, b3, W3)


@jax.jit
def kernel(x, edge_index, W1, b1, W2, b2, W3, b3):
    ei = edge_index.astype(jnp.int32)
    E = ei.shape[1]
    epw = -(-E // (NW * 128)) * 128
    e_pad = epw * NW
    npad_extra = NPAD - N
    pad = e_pad - E
    src = jnp.concatenate(
        [ei[0], jnp.full((pad,), N, jnp.int32)]).reshape(NW, epw // 128, 128)
    dst = jnp.concatenate(
        [ei[1], N + (jnp.arange(pad, dtype=jnp.int32) % npad_extra)]
    ).reshape(NW, epw // 128, 128)

    xpad = jnp.pad(x, ((0, NPAD - N), (0, 0)))
    zeros = jnp.zeros((NPAD, F), jnp.float32)
    ones = jnp.ones((NPAD, F), jnp.float32)
    b1r, b2r = b1.reshape(1, F), b2.reshape(1, F)
    b3r = b3.reshape(1, W3.shape[1])

    xw1 = _tc_xw1(xpad, W1)
    degp = _sc_deg_pass(ones, dst, zeros)
    dinv16, z1 = _tc_deg_finish(degp, xw1)
    q1 = _sc_pass(z1, src, dst, zeros)
    z2 = _tc_layer_mid(q1, z1, dinv16, b1r, W2, scale_out=True)
    q2 = _sc_pass(z2, src, dst, zeros)
    g3 = _tc_layer2_scale(q2, z2, dinv16, b2r)
    q3 = _sc_pass(g3, src, dst, zeros)
    out = _tc_final(q3, g3, dinv16, b3r, W3)
    return out[:N]


# speedup vs baseline: 1.0527x; 1.0527x over previous
import functools

import jax
import jax.numpy as jnp
from jax import lax
from jax.experimental import pallas as pl
from jax.experimental.pallas import tpu as pltpu
from jax.experimental.pallas import tpu_sc as plsc

N = 10000
NPAD = 10240
NC, NS = 2, 16
NW = NC * NS
F = 16
RPS = NPAD // NS
CW = 5120


def _sc_pass(table, src_r, dst_r, zeros):
    nch = src_r.shape[1]
    mesh = plsc.VectorSubcoreMesh(core_axis_name="c", subcore_axis_name="s")

    @functools.partial(
        pl.kernel,
        out_type=jax.ShapeDtypeStruct((NC, NPAD, F), jnp.float32),
        mesh=mesh,
        compiler_params=pltpu.CompilerParams(use_tc_tiling_on_sc=False),
        scratch_types=[
            pltpu.VMEM((nch, CW), jnp.int32),
            pltpu.VMEM((nch, CW), jnp.int32),
            pltpu.VMEM((CW, F), jnp.float32),
            pltpu.VMEM_SHARED((NPAD, F), jnp.float32),
        ],
    )
    def k(table_hbm, src_hbm, dst_hbm, z_hbm, out_hbm, src_v, dst_v, msg_v, acc):
        c = lax.axis_index("c")
        s = lax.axis_index("s")
        w = s * NC + c
        pltpu.sync_copy(src_hbm.at[w], src_v)
        pltpu.sync_copy(dst_hbm.at[w], dst_v)
        pltpu.sync_copy(z_hbm.at[pl.ds(s * RPS, RPS)], acc.at[pl.ds(s * RPS, RPS)])
        plsc.subcore_barrier()

        @pl.loop(0, nch)
        def _(j):
            pltpu.sync_copy(table_hbm.at[src_v.at[j]], msg_v)
            pltpu.sync_copy(msg_v, acc.at[dst_v.at[j]], add=True)

        plsc.subcore_barrier()
        pltpu.sync_copy(acc.at[pl.ds(s * RPS, RPS)],
                        out_hbm.at[c].at[pl.ds(s * RPS, RPS)])

    return k(table, src_r, dst_r, zeros)


def _sc_deg_pass(ones, dst_r, zeros):
    nch = dst_r.shape[1]
    mesh = plsc.VectorSubcoreMesh(core_axis_name="c", subcore_axis_name="s")

    @functools.partial(
        pl.kernel,
        out_type=jax.ShapeDtypeStruct((NC, NPAD, F), jnp.float32),
        mesh=mesh,
        compiler_params=pltpu.CompilerParams(use_tc_tiling_on_sc=False),
        scratch_types=[
            pltpu.VMEM((nch, CW), jnp.int32),
            pltpu.VMEM((CW, F), jnp.float32),
            pltpu.VMEM_SHARED((NPAD, F), jnp.float32),
        ],
    )
    def k(ones_hbm, dst_hbm, z_hbm, out_hbm, dst_v, ones_v, acc):
        c = lax.axis_index("c")
        s = lax.axis_index("s")
        w = s * NC + c
        pltpu.sync_copy(dst_hbm.at[w], dst_v)
        pltpu.sync_copy(ones_hbm, ones_v)
        pltpu.sync_copy(z_hbm.at[pl.ds(s * RPS, RPS)], acc.at[pl.ds(s * RPS, RPS)])
        plsc.subcore_barrier()

        @pl.loop(0, nch)
        def _(j):
            pltpu.sync_copy(ones_v, acc.at[dst_v.at[j]], add=True)

        plsc.subcore_barrier()
        pltpu.sync_copy(acc.at[pl.ds(s * RPS, RPS)],
                        out_hbm.at[c].at[pl.ds(s * RPS, RPS)])

    return k(ones, dst_r, zeros)


_BLK = 2000
_GRID = N // _BLK


def _row_spec(width):
    return pl.BlockSpec((_BLK, width), lambda i: (i, 0))


def _pair_spec(width):
    return pl.BlockSpec((NC, _BLK, width), lambda i: (0, i, 0))


def _full_spec(a, b):
    return pl.BlockSpec((a, b), lambda i: (0, 0))


def _elu(x):
    return jnp.where(x > 0, x, jnp.exp(jnp.minimum(x, 0.0)) - 1.0)


def _tc_deg_finish(degp, x, W1):
    def body(d_ref, x_ref, w_ref, dinv_ref, z_ref):
        deg = 1.0 + d_ref[0, :, 0:1] + d_ref[1, :, 0:1]
        dinv = lax.rsqrt(deg)
        dinv_ref[...] = jnp.broadcast_to(dinv, (_BLK, F))
        xw = jnp.dot(x_ref[...], w_ref[...], preferred_element_type=jnp.float32)
        z_ref[...] = xw * dinv

    return pl.pallas_call(
        body,
        grid=(_GRID,),
        in_specs=[_pair_spec(F), _row_spec(128), _full_spec(128, F)],
        out_specs=[_row_spec(F), _row_spec(F)],
        out_shape=[jax.ShapeDtypeStruct((N, F), jnp.float32),
                   jax.ShapeDtypeStruct((N, F), jnp.float32)],
    )(degp, x, W1)


def _tc_layer_mid(q, z, dinv16, b, Wn):
    def body(q_ref, z_ref, dinv_ref, b_ref, w_ref, o_ref):
        u = q_ref[0] + q_ref[1] + z_ref[...]
        h = _elu(dinv_ref[...] * u + b_ref[...])
        o = jnp.dot(h, w_ref[...], preferred_element_type=jnp.float32)
        o_ref[...] = o * dinv_ref[...]

    return pl.pallas_call(
        body,
        grid=(_GRID,),
        in_specs=[_pair_spec(F), _row_spec(F), _row_spec(F),
                  _full_spec(1, F), _full_spec(F, F)],
        out_specs=_row_spec(F),
        out_shape=jax.ShapeDtypeStruct((N, F), jnp.float32),
    )(q, z, dinv16, b, Wn)


def _tc_layer2_scale(q, z, dinv16, b):
    def body(q_ref, z_ref, dinv_ref, b_ref, o_ref):
        u = q_ref[0] + q_ref[1] + z_ref[...]
        h = _elu(dinv_ref[...] * u + b_ref[...])
        o_ref[...] = h * dinv_ref[...]

    return pl.pallas_call(
        body,
        grid=(_GRID,),
        in_specs=[_pair_spec(F), _row_spec(F), _row_spec(F), _full_spec(1, F)],
        out_specs=_row_spec(F),
        out_shape=jax.ShapeDtypeStruct((N, F), jnp.float32),
    )(q, z, dinv16, b)


def _tc_final(q, g3, dinv16, b3, W3):
    K = W3.shape[1]

    def body(q_ref, g_ref, dinv_ref, b_ref, w_ref, o_ref):
        v = q_ref[0] + q_ref[1] + g_ref[...]
        m = jnp.dot(v, w_ref[...], preferred_element_type=jnp.float32)
        h = _elu(dinv_ref[:, 0:1] * m + b_ref[...])
        mx = jnp.max(h, axis=1, keepdims=True)
        e = jnp.exp(h - mx)
        lse = jnp.log(jnp.sum(e, axis=1, keepdims=True))
        o_ref[...] = h - mx - lse

    return pl.pallas_call(
        body,
        grid=(_GRID,),
        in_specs=[_pair_spec(F), _row_spec(F), _row_spec(F),
                  _full_spec(1, K), _full_spec(F, K)],
        out_specs=_row_spec(K),
        out_shape=jax.ShapeDtypeStruct((N, K), jnp.float32),
    )(q, g3, dinv16, b3, W3)


@jax.jit
def kernel(x, edge_index, W1, b1, W2, b2, W3, b3):
    ei = edge_index.astype(jnp.int32)
    E = ei.shape[1]
    nch = -(-E // (NW * CW))
    e_pad = nch * CW * NW
    pad = e_pad - E
    src = jnp.concatenate(
        [ei[0], jnp.zeros((pad,), jnp.int32)]).reshape(NW, nch, CW)
    dst = jnp.concatenate(
        [ei[1], N + (jnp.arange(pad, dtype=jnp.int32) % (NPAD - N))]
    ).reshape(NW, nch, CW)

    zeros = jnp.zeros((NPAD, F), jnp.float32)
    ones = jnp.ones((CW, F), jnp.float32)
    b1r, b2r = b1.reshape(1, F), b2.reshape(1, F)
    b3r = b3.reshape(1, W3.shape[1])

    degp = _sc_deg_pass(ones, dst, zeros)
    dinv16, z1 = _tc_deg_finish(degp, x, W1)
    q1 = _sc_pass(z1, src, dst, zeros)
    z2 = _tc_layer_mid(q1, z1, dinv16, b1r, W2)
    q2 = _sc_pass(z2, src, dst, zeros)
    g3 = _tc_layer2_scale(q2, z2, dinv16, b2r)
    q3 = _sc_pass(g3, src, dst, zeros)
    return _tc_final(q3, g3, dinv16, b3r, W3)


# speedup vs baseline: 1.7415x; 1.6544x over previous
import functools

import jax
import jax.numpy as jnp
from jax import lax
from jax.experimental import pallas as pl
from jax.experimental.pallas import tpu as pltpu
from jax.experimental.pallas import tpu_sc as plsc

N = 10000
NPAD = 10240
NC, NS = 2, 16
NW = NC * NS
F = 16
RPS = NPAD // NS
CW = 5120


def _sc_pass(table, src_r, dst_r, zeros):
    nch = src_r.shape[1]
    tl_rows = N // NS
    mesh = plsc.VectorSubcoreMesh(core_axis_name="c", subcore_axis_name="s")

    @functools.partial(
        pl.kernel,
        out_type=jax.ShapeDtypeStruct((NC, NPAD, F), jnp.float32),
        mesh=mesh,
        compiler_params=pltpu.CompilerParams(use_tc_tiling_on_sc=False),
        scratch_types=[
            pltpu.VMEM((nch, CW), jnp.int32),
            pltpu.VMEM((nch, CW), jnp.int32),
            pltpu.VMEM((CW, F), jnp.float32),
            pltpu.VMEM_SHARED((N, F), jnp.float32),
            pltpu.VMEM_SHARED((NPAD, F), jnp.float32),
        ],
    )
    def k(table_hbm, src_hbm, dst_hbm, z_hbm, out_hbm,
          src_v, dst_v, msg_v, tbl, acc):
        c = lax.axis_index("c")
        s = lax.axis_index("s")
        w = s * NC + c
        pltpu.sync_copy(src_hbm.at[w], src_v)
        pltpu.sync_copy(dst_hbm.at[w], dst_v)
        pltpu.sync_copy(table_hbm.at[pl.ds(s * tl_rows, tl_rows)],
                        tbl.at[pl.ds(s * tl_rows, tl_rows)])
        pltpu.sync_copy(z_hbm.at[pl.ds(s * RPS, RPS)], acc.at[pl.ds(s * RPS, RPS)])
        plsc.subcore_barrier()

        @pl.loop(0, nch)
        def _(j):
            pltpu.sync_copy(tbl.at[src_v.at[j]], msg_v)
            pltpu.sync_copy(msg_v, acc.at[dst_v.at[j]], add=True)

        plsc.subcore_barrier()
        pltpu.sync_copy(acc.at[pl.ds(s * RPS, RPS)],
                        out_hbm.at[c].at[pl.ds(s * RPS, RPS)])

    return k(table, src_r, dst_r, zeros)


def _sc_deg_pass(ones, dst_r, zeros):
    nch = dst_r.shape[1]
    mesh = plsc.VectorSubcoreMesh(core_axis_name="c", subcore_axis_name="s")

    @functools.partial(
        pl.kernel,
        out_type=jax.ShapeDtypeStruct((NC, NPAD, F), jnp.float32),
        mesh=mesh,
        compiler_params=pltpu.CompilerParams(use_tc_tiling_on_sc=False),
        scratch_types=[
            pltpu.VMEM((nch, CW), jnp.int32),
            pltpu.VMEM((CW, F), jnp.float32),
            pltpu.VMEM_SHARED((NPAD, F), jnp.float32),
        ],
    )
    def k(ones_hbm, dst_hbm, z_hbm, out_hbm, dst_v, ones_v, acc):
        c = lax.axis_index("c")
        s = lax.axis_index("s")
        w = s * NC + c
        pltpu.sync_copy(dst_hbm.at[w], dst_v)
        pltpu.sync_copy(ones_hbm, ones_v)
        pltpu.sync_copy(z_hbm.at[pl.ds(s * RPS, RPS)], acc.at[pl.ds(s * RPS, RPS)])
        plsc.subcore_barrier()

        @pl.loop(0, nch)
        def _(j):
            pltpu.sync_copy(ones_v, acc.at[dst_v.at[j]], add=True)

        plsc.subcore_barrier()
        pltpu.sync_copy(acc.at[pl.ds(s * RPS, RPS)],
                        out_hbm.at[c].at[pl.ds(s * RPS, RPS)])

    return k(ones, dst_r, zeros)


_BLK = 2000
_GRID = N // _BLK


def _row_spec(width):
    return pl.BlockSpec((_BLK, width), lambda i: (i, 0))


def _pair_spec(width):
    return pl.BlockSpec((NC, _BLK, width), lambda i: (0, i, 0))


def _full_spec(a, b):
    return pl.BlockSpec((a, b), lambda i: (0, 0))


def _elu(x):
    return jnp.where(x > 0, x, jnp.exp(jnp.minimum(x, 0.0)) - 1.0)


def _tc_deg_finish(degp, x, W1):
    def body(d_ref, x_ref, w_ref, dinv_ref, z_ref):
        deg = 1.0 + d_ref[0, :, 0:1] + d_ref[1, :, 0:1]
        dinv = lax.rsqrt(deg)
        dinv_ref[...] = jnp.broadcast_to(dinv, (_BLK, F))
        xw = jnp.dot(x_ref[...], w_ref[...], preferred_element_type=jnp.float32)
        z_ref[...] = xw * dinv

    return pl.pallas_call(
        body,
        grid=(_GRID,),
        in_specs=[_pair_spec(F), _row_spec(128), _full_spec(128, F)],
        out_specs=[_row_spec(F), _row_spec(F)],
        out_shape=[jax.ShapeDtypeStruct((N, F), jnp.float32),
                   jax.ShapeDtypeStruct((N, F), jnp.float32)],
    )(degp, x, W1)


def _tc_layer_mid(q, z, dinv16, b, Wn):
    def body(q_ref, z_ref, dinv_ref, b_ref, w_ref, o_ref):
        u = q_ref[0] + q_ref[1] + z_ref[...]
        h = _elu(dinv_ref[...] * u + b_ref[...])
        o = jnp.dot(h, w_ref[...], preferred_element_type=jnp.float32)
        o_ref[...] = o * dinv_ref[...]

    return pl.pallas_call(
        body,
        grid=(_GRID,),
        in_specs=[_pair_spec(F), _row_spec(F), _row_spec(F),
                  _full_spec(1, F), _full_spec(F, F)],
        out_specs=_row_spec(F),
        out_shape=jax.ShapeDtypeStruct((N, F), jnp.float32),
    )(q, z, dinv16, b, Wn)


def _tc_layer2_scale(q, z, dinv16, b):
    def body(q_ref, z_ref, dinv_ref, b_ref, o_ref):
        u = q_ref[0] + q_ref[1] + z_ref[...]
        h = _elu(dinv_ref[...] * u + b_ref[...])
        o_ref[...] = h * dinv_ref[...]

    return pl.pallas_call(
        body,
        grid=(_GRID,),
        in_specs=[_pair_spec(F), _row_spec(F), _row_spec(F), _full_spec(1, F)],
        out_specs=_row_spec(F),
        out_shape=jax.ShapeDtypeStruct((N, F), jnp.float32),
    )(q, z, dinv16, b)


def _tc_final(q, g3, dinv16, b3, W3):
    K = W3.shape[1]

    def body(q_ref, g_ref, dinv_ref, b_ref, w_ref, o_ref):
        v = q_ref[0] + q_ref[1] + g_ref[...]
        m = jnp.dot(v, w_ref[...], preferred_element_type=jnp.float32)
        h = _elu(dinv_ref[:, 0:1] * m + b_ref[...])
        mx = jnp.max(h, axis=1, keepdims=True)
        e = jnp.exp(h - mx)
        lse = jnp.log(jnp.sum(e, axis=1, keepdims=True))
        o_ref[...] = h - mx - lse

    return pl.pallas_call(
        body,
        grid=(_GRID,),
        in_specs=[_pair_spec(F), _row_spec(F), _row_spec(F),
                  _full_spec(1, K), _full_spec(F, K)],
        out_specs=_row_spec(K),
        out_shape=jax.ShapeDtypeStruct((N, K), jnp.float32),
    )(q, g3, dinv16, b3, W3)


@jax.jit
def kernel(x, edge_index, W1, b1, W2, b2, W3, b3):
    ei = edge_index.astype(jnp.int32)
    E = ei.shape[1]
    nch = -(-E // (NW * CW))
    e_pad = nch * CW * NW
    pad = e_pad - E
    src = jnp.concatenate(
        [ei[0], jnp.zeros((pad,), jnp.int32)]).reshape(NW, nch, CW)
    dst = jnp.concatenate(
        [ei[1], N + (jnp.arange(pad, dtype=jnp.int32) % (NPAD - N))]
    ).reshape(NW, nch, CW)

    zeros = jnp.zeros((NPAD, F), jnp.float32)
    ones = jnp.ones((CW, F), jnp.float32)
    b1r, b2r = b1.reshape(1, F), b2.reshape(1, F)
    b3r = b3.reshape(1, W3.shape[1])

    degp = _sc_deg_pass(ones, dst, zeros)
    dinv16, z1 = _tc_deg_finish(degp, x, W1)
    q1 = _sc_pass(z1, src, dst, zeros)
    z2 = _tc_layer_mid(q1, z1, dinv16, b1r, W2)
    q2 = _sc_pass(z2, src, dst, zeros)
    g3 = _tc_layer2_scale(q2, z2, dinv16, b2r)
    q3 = _sc_pass(g3, src, dst, zeros)
    return _tc_final(q3, g3, dinv16, b3r, W3)
